# Initial kernel scaffold; baseline (speedup 1.0000x reference)
import jax
import jax.numpy as jnp
from jax.experimental import pallas as pl


def kernel(data, edge_index, d, fw_params, bw_params, find_params):
    raise NotImplementedError("write your pallas kernel here")


# speedup vs baseline: 1.9079x; 1.9079x over previous
import jax
import jax.numpy as jnp
from jax import lax
from jax.experimental import pallas as pl
from jax.experimental.pallas import tpu as pltpu
from jax.experimental.pallas import tpu_sc as plsc

_EMB = 5
_NN = 8
_NE = 16
_NEG = 0.01
_L = 16
_IN_DIMS = [1, 5, 5]


def _sig(x):
    return 1.0 / (1.0 + jnp.exp(-x))


def _lrelu(x):
    return jnp.where(x >= 0, x, _NEG * x)


def _pad16(n):
    return (n + 15) & ~15


def kernel(data, edge_index, d, fw_params, bw_params, find_params):
    ei = edge_index.astype(jnp.int32)

    segs = []
    offs = {}
    pos = 0

    def add(name, arr):
        nonlocal pos
        arr = arr.reshape(-1).astype(jnp.float32)
        offs[name] = pos
        segs.append(arr)
        n = arr.shape[0]
        padded = _pad16(n)
        if padded > n:
            segs.append(jnp.zeros((padded - n,), jnp.float32))
        pos += padded

    add("data", data)
    add("d", d)
    for li, params in enumerate(list(fw_params) + list(bw_params)):
        ws, wm, wg, b = params
        add(f"ws{li}", ws)
        add(f"wm{li}", wm)
        add(f"wg{li}", wg)
        add(f"b{li}", b)
    for li, (W, b) in enumerate(find_params[:3]):
        add(f"W{li}", W)
        add(f"Wb{li}", b)
    W4, b4 = find_params[3]
    add("W4T", W4.T)
    add("b4", b4)
    P = jnp.concatenate(segs)
    n_p = P.shape[0]

    mesh = plsc.VectorSubcoreMesh(core_axis_name="c", subcore_axis_name="s")

    scratch = [
        pltpu.VMEM((n_p,), jnp.float32),
        pltpu.VMEM((2, _L), jnp.int32),
        pltpu.VMEM((_EMB, _L), jnp.float32),
        pltpu.VMEM((_EMB, _L), jnp.float32),
        pltpu.VMEM((_EMB * _L,), jnp.float32),
        pltpu.VMEM((112,), jnp.float32),
        pltpu.VMEM((128,), jnp.float32),
        pltpu.VMEM((128,), jnp.float32),
        pltpu.VMEM((64,), jnp.float32),
        pltpu.VMEM((_L,), jnp.float32),
        pltpu.SemaphoreType.DMA,
    ]

    def body(p_hbm, e_hbm, out_ref, p_ref, e_ref, x_ref, xx_ref, agg_ref,
             v_ref, h1_ref, h2_ref, h3_ref, outv_ref, sem):
        is_worker = (lax.axis_index("c") == 0) & (lax.axis_index("s") == 0)

        @pl.when(is_worker)
        def _():
            iota = lax.iota(jnp.int32, _L)
            zero = jnp.zeros((_L,), jnp.float32)

            h1 = pltpu.async_copy(p_hbm, p_ref, sem)
            h2 = pltpu.async_copy(e_hbm, e_ref, sem)
            h1.wait()
            h2.wait()

            def bc(off):
                row = p_ref[pl.ds((off // _L) * _L, _L)]
                return jnp.full((_L,), row[off % _L], jnp.float32)

            src = e_ref[0, :]
            dst = e_ref[1, :]

            x0 = jnp.where(iota < _NN,
                           plsc.load_gather(p_ref, [offs["data"] + (iota & (_NN - 1))]),
                           0.0)
            x_ref[0, :] = x0
            xx_ref[0, :] = x0
            for f in range(1, _EMB):
                x_ref[f, :] = zero
                xx_ref[f, :] = zero

            def gated(xr, li, s_vec, t_vec):
                in_dim = _IN_DIMS[li % 3]
                ws_o, wm_o, wg_o, b_o = (offs[f"ws{li}"], offs[f"wm{li}"],
                                         offs[f"wg{li}"], offs[f"b{li}"])
                g = [plsc.load_gather(xr, [jnp.full((_L,), f, jnp.int32), s_vec])
                     for f in range(in_dim)]
                glin = g[0] * bc(wg_o)
                for f in range(1, in_dim):
                    glin = glin + g[f] * bc(wg_o + f)
                gate = _sig(glin)
                for k in range(_EMB):
                    agg_ref[pl.ds(k * _L, _L)] = zero
                for k in range(_EMB):
                    msg = g[0] * bc(wm_o + k)
                    for f in range(1, in_dim):
                        msg = msg + g[f] * bc(wm_o + f * _EMB + k)
                    plsc.addupdate_scatter(agg_ref, [t_vec + k * _L],
                                           gate * msg)
                xs = [xr[f, :] for f in range(in_dim)]
                new = []
                for k in range(_EMB):
                    acc = agg_ref[pl.ds(k * _L, _L)] + bc(b_o + k)
                    for f in range(in_dim):
                        acc = acc + xs[f] * bc(ws_o + f * _EMB + k)
                    new.append(_lrelu(acc))
                for k in range(_EMB):
                    xr[k, :] = new[k]

            for l in range(3):
                gated(x_ref, l, src, dst)
            for l in range(3):
                gated(xx_ref, 3 + l, dst, src)

            lane_mask = iota < _NN
            for f in range(_EMB):
                plsc.store_scatter(v_ref, [iota * _EMB + f], x_ref[f, :],
                                   mask=lane_mask)
                plsc.store_scatter(v_ref, [40 + iota * _EMB + f], xx_ref[f, :],
                                   mask=lane_mask)
            d_o = offs["d"]
            v_ref[pl.ds(80, _L)] = p_ref[pl.ds(d_o, _L)]
            v_ref[pl.ds(96, _L)] = jnp.where(
                iota < 8,
                plsc.load_gather(p_ref, [d_o + jnp.minimum(iota + 16, 23)]),
                0.0)

            def dense(src_ref, w_o, b_o, in_dim, out_dim, dst_ref, act):
                nchunk = out_dim // _L
                accs = [p_ref[pl.ds(b_o + _L * c, _L)] for c in range(nchunk)]
                for i in range(in_dim):
                    row = src_ref[pl.ds((i // _L) * _L, _L)]
                    bv = jnp.full((_L,), row[i % _L], jnp.float32)
                    for c in range(nchunk):
                        accs[c] = accs[c] + bv * p_ref[
                            pl.ds(w_o + i * out_dim + _L * c, _L)]
                for c in range(nchunk):
                    dst_ref[pl.ds(_L * c, _L)] = act(accs[c])

            dense(v_ref, offs["W0"], offs["Wb0"], 104, 128, h1_ref, _lrelu)
            dense(h1_ref, offs["W1"], offs["Wb1"], 128, 128, h2_ref, _lrelu)
            dense(h2_ref, offs["W2"], offs["Wb2"], 128, 64, h3_ref, _lrelu)

            outs = []
            for j in range(2):
                t = zero
                for q in range(4):
                    t = t + (h3_ref[pl.ds(_L * q, _L)]
                             * p_ref[pl.ds(offs["W4T"] + j * 64 + _L * q, _L)])
                tj = jnp.sum(t)
                outs.append(_sig(jnp.full((_L,), tj) + bc(offs["b4"] + j)))
            outv_ref[...] = 0.5 * outs[0] + 0.5 * outs[1]
            pltpu.sync_copy(outv_ref.at[pl.ds(0, _NN)], out_ref)

    run = pl.kernel(
        body,
        out_type=jax.ShapeDtypeStruct((_NN,), jnp.float32),
        mesh=mesh,
        scratch_types=scratch,
        compiler_params=pltpu.CompilerParams(needs_layout_passes=False),
    )
    out = run(P, ei)
    return out[0]


# speedup vs baseline: 2.1129x; 1.1074x over previous
import jax
import jax.numpy as jnp
from jax import lax
from jax.experimental import pallas as pl
from jax.experimental.pallas import tpu as pltpu
from jax.experimental.pallas import tpu_sc as plsc

_EMB = 5
_NN = 8
_NE = 16
_NEG = 0.01
_L = 16
_IN_DIMS = [1, 5, 5]


def _sig(x):
    return 1.0 / (1.0 + jnp.exp(-x))


def _lrelu(x):
    return jnp.where(x >= 0, x, _NEG * x)


def _pad16(n):
    return (n + 15) & ~15


class _Packer:
    def __init__(self):
        self.segs = []
        self.offs = {}
        self.pos = 0

    def add(self, name, arr):
        arr = arr.reshape(-1).astype(jnp.float32)
        self.offs[name] = self.pos
        self.segs.append(arr)
        n = arr.shape[0]
        padded = _pad16(n)
        if padded > n:
            self.segs.append(jnp.zeros((padded - n,), jnp.float32))
        self.pos += padded

    def concat(self):
        return jnp.concatenate(self.segs)


def kernel(data, edge_index, d, fw_params, bw_params, find_params):
    ei = edge_index.astype(jnp.int32)

    pg = _Packer()
    pg.add("data", data)
    pg.add("d", d)
    for li, params in enumerate(list(fw_params) + list(bw_params)):
        ws, wm, wg, b = params
        pg.add(f"ws{li}", ws)
        pg.add(f"wm{li}", wm)
        pg.add(f"wg{li}", wg)
        pg.add(f"b{li}", b)
    PG = pg.concat()
    og = pg.offs

    pm = _Packer()
    for li, (W, b) in enumerate(find_params[:3]):
        pm.add(f"W{li}", W)
        pm.add(f"Wb{li}", b)
    W4, b4 = find_params[3]
    pm.add("W4T", W4.T)
    pm.add("b4", b4)
    PM = pm.concat()
    om = pm.offs

    mesh = plsc.VectorSubcoreMesh(core_axis_name="c", subcore_axis_name="s")

    scratch = [
        pltpu.VMEM((PG.shape[0],), jnp.float32),
        pltpu.VMEM((PM.shape[0],), jnp.float32),
        pltpu.VMEM((2, _L), jnp.int32),
        pltpu.VMEM((_EMB, _L), jnp.float32),
        pltpu.VMEM((_EMB, _L), jnp.float32),
        pltpu.VMEM((_EMB * _L,), jnp.float32),
        pltpu.VMEM((112,), jnp.float32),
        pltpu.VMEM((128,), jnp.float32),
        pltpu.VMEM((128,), jnp.float32),
        pltpu.VMEM((64,), jnp.float32),
        pltpu.VMEM((_L,), jnp.float32),
        pltpu.SemaphoreType.DMA,
        pltpu.SemaphoreType.DMA,
    ]

    def body(pg_hbm, pm_hbm, e_hbm, out_ref, pg_ref, pm_ref, e_ref, x_ref,
             xx_ref, agg_ref, v_ref, h1_ref, h2_ref, h3_ref, outv_ref,
             sem_g, sem_m):
        is_worker = (lax.axis_index("c") == 0) & (lax.axis_index("s") == 0)

        @pl.when(is_worker)
        def _():
            iota = lax.iota(jnp.int32, _L)
            zero = jnp.zeros((_L,), jnp.float32)

            hm = pltpu.async_copy(pm_hbm, pm_ref, sem_m)
            hg = pltpu.async_copy(pg_hbm, pg_ref, sem_g)
            he = pltpu.async_copy(e_hbm, e_ref, sem_g)
            hg.wait()
            he.wait()

            def rows_of(ref, off, count):
                n_rows = (count + _L - 1) // _L
                return [ref[pl.ds(off + r * _L, _L)] for r in range(n_rows)]

            def splat(rows, i):
                return jnp.full((_L,), rows[i // _L][i % _L], jnp.float32)

            src = e_ref[0, :]
            dst = e_ref[1, :]

            x0 = jnp.where(
                iota < _NN,
                plsc.load_gather(pg_ref, [og["data"] + (iota & (_NN - 1))]),
                0.0)
            x_ref[0, :] = x0
            xx_ref[0, :] = x0
            for f in range(1, _EMB):
                x_ref[f, :] = zero
                xx_ref[f, :] = zero

            def gated(xr, li, s_vec, t_vec):
                in_dim = _IN_DIMS[li % 3]
                ws_r = rows_of(pg_ref, og[f"ws{li}"], in_dim * _EMB)
                wm_r = rows_of(pg_ref, og[f"wm{li}"], in_dim * _EMB)
                wg_r = rows_of(pg_ref, og[f"wg{li}"], in_dim)
                b_r = rows_of(pg_ref, og[f"b{li}"], _EMB)
                g = [plsc.load_gather(xr, [jnp.full((_L,), f, jnp.int32), s_vec])
                     for f in range(in_dim)]
                glin = g[0] * splat(wg_r, 0)
                for f in range(1, in_dim):
                    glin = glin + g[f] * splat(wg_r, f)
                gate = _sig(glin)
                for k in range(_EMB):
                    agg_ref[pl.ds(k * _L, _L)] = zero
                for k in range(_EMB):
                    msg = g[0] * splat(wm_r, k)
                    for f in range(1, in_dim):
                        msg = msg + g[f] * splat(wm_r, f * _EMB + k)
                    plsc.addupdate_scatter(agg_ref, [t_vec + k * _L],
                                           gate * msg)
                xs = [xr[f, :] for f in range(in_dim)]
                new = []
                for k in range(_EMB):
                    acc = agg_ref[pl.ds(k * _L, _L)] + splat(b_r, k)
                    for f in range(in_dim):
                        acc = acc + xs[f] * splat(ws_r, f * _EMB + k)
                    new.append(_lrelu(acc))
                for k in range(_EMB):
                    xr[k, :] = new[k]

            for l in range(3):
                gated(x_ref, l, src, dst)
            for l in range(3):
                gated(xx_ref, 3 + l, dst, src)

            lane_mask = iota < _NN
            for f in range(_EMB):
                plsc.store_scatter(v_ref, [iota * _EMB + f], x_ref[f, :],
                                   mask=lane_mask)
                plsc.store_scatter(v_ref, [40 + iota * _EMB + f], xx_ref[f, :],
                                   mask=lane_mask)
            d_o = og["d"]
            v_ref[pl.ds(80, _L)] = pg_ref[pl.ds(d_o, _L)]
            v_ref[pl.ds(96, _L)] = jnp.where(
                iota < 8,
                plsc.load_gather(pg_ref, [d_o + jnp.minimum(iota + 16, 23)]),
                0.0)

            hm.wait()

            def dense(src_ref, w_o, b_o, in_dim, out_dim, dst_ref, act):
                nchunk = out_dim // _L
                accs = [pm_ref[pl.ds(b_o + _L * c, _L)] for c in range(nchunk)]
                for blk in range(0, in_dim, _L):
                    row = src_ref[pl.ds(blk, _L)]
                    for lane in range(min(_L, in_dim - blk)):
                        i = blk + lane
                        bv = jnp.full((_L,), row[lane], jnp.float32)
                        for c in range(nchunk):
                            accs[c] = accs[c] + bv * pm_ref[
                                pl.ds(w_o + i * out_dim + _L * c, _L)]
                for c in range(nchunk):
                    dst_ref[pl.ds(_L * c, _L)] = act(accs[c])

            dense(v_ref, om["W0"], om["Wb0"], 104, 128, h1_ref, _lrelu)
            dense(h1_ref, om["W1"], om["Wb1"], 128, 128, h2_ref, _lrelu)
            dense(h2_ref, om["W2"], om["Wb2"], 128, 64, h3_ref, _lrelu)

            b4_r = rows_of(pm_ref, om["b4"], 2)
            outs = []
            for j in range(2):
                t = zero
                for q in range(4):
                    t = t + (h3_ref[pl.ds(_L * q, _L)]
                             * pm_ref[pl.ds(om["W4T"] + j * 64 + _L * q, _L)])
                tj = jnp.sum(t)
                outs.append(_sig(jnp.full((_L,), tj) + splat(b4_r, j)))
            outv_ref[...] = 0.5 * outs[0] + 0.5 * outs[1]
            pltpu.sync_copy(outv_ref.at[pl.ds(0, _NN)], out_ref)

    run = pl.kernel(
        body,
        out_type=jax.ShapeDtypeStruct((_NN,), jnp.float32),
        mesh=mesh,
        scratch_types=scratch,
        compiler_params=pltpu.CompilerParams(needs_layout_passes=False),
    )
    out = run(PG, PM, ei)
    return out[0]


# speedup vs baseline: 2.1915x; 1.0372x over previous
import jax
import jax.numpy as jnp
from jax import lax
from jax.experimental import pallas as pl
from jax.experimental.pallas import tpu as pltpu
from jax.experimental.pallas import tpu_sc as plsc

_EMB = 5
_NN = 8
_NE = 16
_NEG = 0.01
_L = 16
_IN_DIMS = [1, 5, 5]


def _sig(x):
    return 1.0 / (1.0 + jnp.exp(-x))


def _lrelu(x):
    return jnp.where(x >= 0, x, _NEG * x)


def _pad16(n):
    return (n + 15) & ~15


class _Packer:
    def __init__(self):
        self.segs = []
        self.offs = {}
        self.pos = 0

    def add(self, name, arr):
        arr = arr.reshape(-1).astype(jnp.float32)
        self.offs[name] = self.pos
        self.segs.append(arr)
        n = arr.shape[0]
        padded = _pad16(n)
        if padded > n:
            self.segs.append(jnp.zeros((padded - n,), jnp.float32))
        self.pos += padded

    def concat(self):
        return jnp.concatenate(self.segs)


def kernel(data, edge_index, d, fw_params, bw_params, find_params):
    ei = edge_index.astype(jnp.int32)

    pg = _Packer()
    pg.add("data", data)
    pg.add("d", d)
    for li, params in enumerate(list(fw_params) + list(bw_params)):
        ws, wm, wg, b = params
        pg.add(f"ws{li}", ws)
        pg.add(f"wm{li}", wm)
        pg.add(f"wg{li}", wg)
        pg.add(f"b{li}", b)
    PG = pg.concat()
    og = pg.offs

    pm = _Packer()
    for li, (W, b) in enumerate(find_params[:3]):
        pm.add(f"W{li}", W)
        pm.add(f"Wb{li}", b)
    W4, b4 = find_params[3]
    pm.add("W4T", W4.T)
    pm.add("b4", b4)
    PM = pm.concat()
    om = pm.offs

    mesh = plsc.VectorSubcoreMesh(core_axis_name="c", subcore_axis_name="s",
                                  num_cores=1, num_subcores=1)

    scratch = [
        pltpu.VMEM((PG.shape[0],), jnp.float32),
        pltpu.VMEM((PM.shape[0],), jnp.float32),
        pltpu.VMEM((2, _L), jnp.int32),
        pltpu.VMEM((_EMB, _L), jnp.float32),
        pltpu.VMEM((_EMB, _L), jnp.float32),
        pltpu.VMEM((_EMB * _L,), jnp.float32),
        pltpu.VMEM((112,), jnp.float32),
        pltpu.VMEM((128,), jnp.float32),
        pltpu.VMEM((128,), jnp.float32),
        pltpu.VMEM((64,), jnp.float32),
        pltpu.VMEM((_L,), jnp.float32),
        pltpu.SemaphoreType.DMA,
        pltpu.SemaphoreType.DMA,
    ]

    def body(pg_hbm, pm_hbm, e_hbm, out_ref, pg_ref, pm_ref, e_ref, x_ref,
             xx_ref, agg_ref, v_ref, h1_ref, h2_ref, h3_ref, outv_ref,
             sem_g, sem_m):
        @pl.when(lax.axis_index("c") == 0)
        def _():
            iota = lax.iota(jnp.int32, _L)
            zero = jnp.zeros((_L,), jnp.float32)

            hm = pltpu.async_copy(pm_hbm, pm_ref, sem_m)
            hg = pltpu.async_copy(pg_hbm, pg_ref, sem_g)
            he = pltpu.async_copy(e_hbm, e_ref, sem_g)
            hg.wait()
            he.wait()

            def rows_of(ref, off, count):
                n_rows = (count + _L - 1) // _L
                return [ref[pl.ds(off + r * _L, _L)] for r in range(n_rows)]

            def splat(rows, i):
                return jnp.full((_L,), rows[i // _L][i % _L], jnp.float32)

            src = e_ref[0, :]
            dst = e_ref[1, :]

            x0 = jnp.where(
                iota < _NN,
                plsc.load_gather(pg_ref, [og["data"] + (iota & (_NN - 1))]),
                0.0)
            x_ref[0, :] = x0
            xx_ref[0, :] = x0
            for f in range(1, _EMB):
                x_ref[f, :] = zero
                xx_ref[f, :] = zero

            def gated(xr, li, s_vec, t_vec):
                in_dim = _IN_DIMS[li % 3]
                ws_r = rows_of(pg_ref, og[f"ws{li}"], in_dim * _EMB)
                wm_r = rows_of(pg_ref, og[f"wm{li}"], in_dim * _EMB)
                wg_r = rows_of(pg_ref, og[f"wg{li}"], in_dim)
                b_r = rows_of(pg_ref, og[f"b{li}"], _EMB)
                g = [plsc.load_gather(xr, [jnp.full((_L,), f, jnp.int32), s_vec])
                     for f in range(in_dim)]
                glin = g[0] * splat(wg_r, 0)
                for f in range(1, in_dim):
                    glin = glin + g[f] * splat(wg_r, f)
                gate = _sig(glin)
                for k in range(_EMB):
                    agg_ref[pl.ds(k * _L, _L)] = zero
                for k in range(_EMB):
                    msg = g[0] * splat(wm_r, k)
                    for f in range(1, in_dim):
                        msg = msg + g[f] * splat(wm_r, f * _EMB + k)
                    plsc.addupdate_scatter(agg_ref, [t_vec + k * _L],
                                           gate * msg)
                xs = [xr[f, :] for f in range(in_dim)]
                new = []
                for k in range(_EMB):
                    acc = agg_ref[pl.ds(k * _L, _L)] + splat(b_r, k)
                    for f in range(in_dim):
                        acc = acc + xs[f] * splat(ws_r, f * _EMB + k)
                    new.append(_lrelu(acc))
                for k in range(_EMB):
                    xr[k, :] = new[k]

            for l in range(3):
                gated(x_ref, l, src, dst)
            for l in range(3):
                gated(xx_ref, 3 + l, dst, src)

            lane_mask = iota < _NN
            for f in range(_EMB):
                plsc.store_scatter(v_ref, [iota * _EMB + f], x_ref[f, :],
                                   mask=lane_mask)
                plsc.store_scatter(v_ref, [40 + iota * _EMB + f], xx_ref[f, :],
                                   mask=lane_mask)
            d_o = og["d"]
            v_ref[pl.ds(80, _L)] = pg_ref[pl.ds(d_o, _L)]
            v_ref[pl.ds(96, _L)] = jnp.where(
                iota < 8,
                plsc.load_gather(pg_ref, [d_o + jnp.minimum(iota + 16, 23)]),
                0.0)

            hm.wait()

            def dense(src_ref, w_o, b_o, in_dim, out_dim, dst_ref, act):
                nchunk = out_dim // _L
                accs = [pm_ref[pl.ds(b_o + _L * c, _L)] for c in range(nchunk)]
                for blk in range(0, in_dim, _L):
                    row = src_ref[pl.ds(blk, _L)]
                    for lane in range(min(_L, in_dim - blk)):
                        i = blk + lane
                        bv = jnp.full((_L,), row[lane], jnp.float32)
                        for c in range(nchunk):
                            accs[c] = accs[c] + bv * pm_ref[
                                pl.ds(w_o + i * out_dim + _L * c, _L)]
                for c in range(nchunk):
                    dst_ref[pl.ds(_L * c, _L)] = act(accs[c])

            dense(v_ref, om["W0"], om["Wb0"], 104, 128, h1_ref, _lrelu)
            dense(h1_ref, om["W1"], om["Wb1"], 128, 128, h2_ref, _lrelu)
            dense(h2_ref, om["W2"], om["Wb2"], 128, 64, h3_ref, _lrelu)

            b4_r = rows_of(pm_ref, om["b4"], 2)
            outs = []
            for j in range(2):
                t = zero
                for q in range(4):
                    t = t + (h3_ref[pl.ds(_L * q, _L)]
                             * pm_ref[pl.ds(om["W4T"] + j * 64 + _L * q, _L)])
                tj = jnp.sum(t)
                outs.append(_sig(jnp.full((_L,), tj) + splat(b4_r, j)))
            outv_ref[...] = 0.5 * outs[0] + 0.5 * outs[1]
            pltpu.sync_copy(outv_ref.at[pl.ds(0, _NN)], out_ref)

    run = pl.kernel(
        body,
        out_type=jax.ShapeDtypeStruct((_NN,), jnp.float32),
        mesh=mesh,
        scratch_types=scratch,
        compiler_params=pltpu.CompilerParams(needs_layout_passes=False),
    )
    out = run(PG, PM, ei)
    return out[0]
